# baseline (speedup 1.0000x reference)
import jax
import jax.numpy as jnp
from jax.experimental import pallas as pl
from jax.experimental.pallas import tpu as pltpu


def _q2c_kernel(h_ref, s_ref, o_ref):
    s = s_ref[0]
    h = h_ref[0]
    z1 = jnp.max(s, axis=-1, keepdims=True)
    m = jnp.max(z1, axis=0, keepdims=True)
    e = jnp.exp(z1 - m)
    denom = jnp.sum(e, axis=0, keepdims=True)
    p = jnp.sum(e * h, axis=0, keepdims=True)
    pooled = p / denom
    o_ref[0] = jnp.broadcast_to(pooled, h.shape)


def kernel(h, s):
    B, T, D = h.shape
    J = s.shape[-1]
    return pl.pallas_call(
        _q2c_kernel,
        grid=(B,),
        in_specs=[
            pl.BlockSpec((1, T, D), lambda b: (b, 0, 0)),
            pl.BlockSpec((1, T, J), lambda b: (b, 0, 0)),
        ],
        out_specs=pl.BlockSpec((1, T, D), lambda b: (b, 0, 0)),
        out_shape=jax.ShapeDtypeStruct(h.shape, h.dtype),
        compiler_params=pltpu.CompilerParams(
            dimension_semantics=("arbitrary",),
        ),
    )(h, s)


# speedup vs baseline: 1.0050x; 1.0050x over previous
import jax
import jax.numpy as jnp
from jax.experimental import pallas as pl
from jax.experimental.pallas import tpu as pltpu

_NBUF = 4


def _make_body(B, T, D):
    def body(h_ref, s_ref, o_hbm, o_buf, o_sem):
        b = pl.program_id(0)
        slot = jax.lax.rem(b, _NBUF)

        s = s_ref[0]
        h = h_ref[0]
        z1 = jnp.max(s, axis=-1, keepdims=True)
        m = jnp.max(z1, axis=0, keepdims=True)
        e = jnp.exp(z1 - m)
        denom = jnp.sum(e, axis=0, keepdims=True)
        p = jnp.sum(e * h, axis=0, keepdims=True)
        pooled = p / denom

        @pl.when(b >= _NBUF)
        def _():
            pltpu.make_async_copy(
                o_buf.at[slot], o_hbm.at[b - _NBUF], o_sem.at[slot]
            ).wait()

        o_buf[pl.ds(slot, 1)] = jnp.broadcast_to(pooled, (1, T, D))
        pltpu.make_async_copy(
            o_buf.at[slot], o_hbm.at[b], o_sem.at[slot]
        ).start(priority=1)

        @pl.when(b == B - 1)
        def _():
            for k in range(_NBUF):
                step = B - _NBUF + k
                pltpu.make_async_copy(
                    o_buf.at[step % _NBUF], o_hbm.at[step], o_sem.at[step % _NBUF]
                ).wait()

    return body


def kernel(h, s):
    B, T, D = h.shape
    J = s.shape[-1]
    return pl.pallas_call(
        _make_body(B, T, D),
        grid=(B,),
        in_specs=[
            pl.BlockSpec((1, T, D), lambda b: (b, 0, 0)),
            pl.BlockSpec((1, T, J), lambda b: (b, 0, 0)),
        ],
        out_specs=pl.BlockSpec(memory_space=pl.ANY),
        out_shape=jax.ShapeDtypeStruct(h.shape, h.dtype),
        scratch_shapes=[
            pltpu.VMEM((_NBUF, T, D), jnp.float32),
            pltpu.SemaphoreType.DMA((_NBUF,)),
        ],
        compiler_params=pltpu.CompilerParams(
            dimension_semantics=("arbitrary",),
        ),
    )(h, s)
